# baseline (speedup 1.0000x reference)
import functools

import jax
import jax.numpy as jnp
from jax import lax
from jax.experimental import pallas as pl
from jax.experimental.pallas import tpu as pltpu
from jax.experimental.pallas import tpu_sc as plsc

B = 16384
D = 32
NC = 2
NS = 16
NW = NC * NS
BPW = B // NW
NCHUNK = 4
CH = BPW // NCHUNK
GPC = CH // 16


def _fpmc_body(u_hbm, l_hbm, n_hbm, ui_hbm, iu_hbm, li_hbm, il_hbm, out_hbm,
               u_v, l_v, n_v, ui_v, iu_v, li_v, il_v, out_v, sem):
    wid = lax.axis_index("s") * NC + lax.axis_index("c")

    pltpu.sync_copy(u_hbm.at[wid], u_v)
    pltpu.sync_copy(l_hbm.at[wid], l_v)
    pltpu.sync_copy(n_hbm.at[wid], n_v)

    for c in range(NCHUNK):
        copies = []
        for j in range(D):
            copies.append(pltpu.async_copy(
                ui_hbm.at[j].at[u_v.at[c]], ui_v.at[j], sem))
            copies.append(pltpu.async_copy(
                iu_hbm.at[j].at[n_v.at[c]], iu_v.at[j], sem))
            copies.append(pltpu.async_copy(
                li_hbm.at[j].at[l_v.at[c]], li_v.at[j], sem))
            copies.append(pltpu.async_copy(
                il_hbm.at[j].at[n_v.at[c]], il_v.at[j], sem))
        for cp in copies:
            cp.wait()

        def group(g, carry, c=c):
            o = pl.multiple_of(g * 16, 16)
            acc = jnp.zeros((16,), jnp.float32)
            for j in range(D):
                acc = (acc + ui_v[j, pl.ds(o, 16)] * iu_v[j, pl.ds(o, 16)]
                       + il_v[j, pl.ds(o, 16)] * li_v[j, pl.ds(o, 16)])
            sig = 1.0 / (1.0 + jnp.exp(-acc))
            out_v[pl.ds(pl.multiple_of(c * CH + g * 16, 16), 16)] = sig
            return carry

        lax.fori_loop(0, GPC, group, 0)

    base = pl.multiple_of(wid * BPW, BPW)
    pltpu.sync_copy(out_v, out_hbm.at[pl.ds(base, BPW)])


_fpmc = functools.partial(
    pl.kernel,
    out_type=jax.ShapeDtypeStruct((B,), jnp.float32),
    mesh=plsc.VectorSubcoreMesh(core_axis_name="c", subcore_axis_name="s"),
    compiler_params=pltpu.CompilerParams(
        needs_layout_passes=False, use_tc_tiling_on_sc=False),
    scratch_types=[
        pltpu.VMEM((NCHUNK, CH), jnp.int32),
        pltpu.VMEM((NCHUNK, CH), jnp.int32),
        pltpu.VMEM((NCHUNK, CH), jnp.int32),
        pltpu.VMEM((D, CH), jnp.float32),
        pltpu.VMEM((D, CH), jnp.float32),
        pltpu.VMEM((D, CH), jnp.float32),
        pltpu.VMEM((D, CH), jnp.float32),
        pltpu.VMEM((BPW,), jnp.float32),
        pltpu.SemaphoreType.DMA,
    ],
)(_fpmc_body)


def kernel(user_id, item_last_click, next_item, UI, IU, LI, IL):
    u = user_id.reshape(NW, NCHUNK, CH).astype(jnp.int32)
    l = item_last_click.reshape(NW, NCHUNK, CH).astype(jnp.int32)
    n = next_item.reshape(NW, NCHUNK, CH).astype(jnp.int32)
    return _fpmc(u, l, n,
                 jnp.swapaxes(UI, 0, 1), jnp.swapaxes(IU, 0, 1),
                 jnp.swapaxes(LI, 0, 1), jnp.swapaxes(IL, 0, 1))
